# baseline (speedup 1.0000x reference)
import jax
import jax.numpy as jnp
from jax.experimental import pallas as pl
from jax.experimental.pallas import tpu as pltpu

_N = 25
_E = 8
_D = 16
_TLM = 5
_NBR = 3
_HID = 32
_TOUT = 3
_T = 8
_B = 1024
_IN = _TLM * _NBR * _D


def _moe_body(xf, w1n, b1r, w2n, b2r, tot_ref, eidx_ref,
              xbf_ref, targ_ref, w1t_ref, w2b_ref):
    p = pl.program_id(0)
    f32 = jnp.float32
    bf = jnp.bfloat16
    B = eidx_ref.shape[-1]

    @pl.when(p == 0)
    def _transpose():
        big = xf[...].T.reshape(_T, _N, _D, B)
        targ_ref[...] = big[2:2 + _TOUT]
        xbf_ref[...] = big[:_TLM + _TOUT - 1].astype(bf)
        w1t_ref[...] = jnp.transpose(w1n[...], (0, 1, 3, 2)).reshape(
            _N, _E * _HID, _IN).astype(bf)
        w2b_ref[...] = jnp.zeros((_N, _E * _D, _E * _HID), bf)
        w2t = jnp.transpose(w2n[...], (0, 1, 3, 2)).astype(bf)
        for e in range(_E):
            w2b_ref[:, _D * e:_D * (e + 1),
                    _HID * e:_HID * (e + 1)] = w2t[:, e]
        tot_ref[0, 0] = f32(0.0)

    @pl.when(p > 0)
    def _compute():
        j = p - 1
        slab = {(tt, nb): xbf_ref[tt, (j + nb + _N - 1) % _N]
                for tt in range(_TLM + _TOUT - 1) for nb in range(_NBR)}
        flat3 = jnp.concatenate(
            [jnp.concatenate([slab[(tc + tl, nb)] for tl in range(_TLM)
                              for nb in range(_NBR)], axis=0)
             for tc in range(_TOUT)], axis=1)

        h = jnp.maximum(
            jnp.dot(w1t_ref[j], flat3, preferred_element_type=f32)
            + b1r[0], 0.0)
        pred = jnp.dot(w2b_ref[j], h.astype(bf),
                       preferred_element_type=f32) + b2r[0]

        targ = jnp.concatenate([targ_ref[tc, j] for tc in range(_TOUT)],
                               axis=1)
        targ_t = jnp.concatenate([targ] * _E, axis=0)
        sq = (pred - targ_t) * (pred - targ_t)

        err = jnp.sum(sq.reshape(_E, _D, sq.shape[-1]), axis=1) * f32(1. / _D)

        minv = jnp.min(err, axis=0, keepdims=True)
        tot_ref[0, 0] = tot_ref[0, 0] + jnp.sum(minv)

        @pl.when(j == _N - 1)
        def _final():
            err_l = err[:, (_TOUT - 1) * B:]
            min_l = minv[:, (_TOUT - 1) * B:]
            ei = jax.lax.broadcasted_iota(jnp.int32, err_l.shape, 0)
            amin = jnp.min(jnp.where(err_l == min_l, ei, _E), axis=0)
            eidx_ref[0, :] = amin
            tot_ref[0, 0] = tot_ref[0, 0] * f32(1.0 / (B * (_N - 1) * 4))


def kernel(x, W1, b1, W2, b2):
    B, T, N, D = x.shape

    xf = x.reshape(B, T * N * D)
    b1r = b1.reshape(N, _E * _HID, 1)
    b2r = b2.reshape(N, _E * _D, 1)

    def _bmap(p):
        return (jnp.maximum(p - 1, 0), 0, 0)

    total, eidx = pl.pallas_call(
        _moe_body,
        grid=(1 + N,),
        in_specs=[
            pl.BlockSpec((B, T * N * D), lambda p: (0, 0)),
            pl.BlockSpec((N, _E, _IN, _HID), lambda p: (0, 0, 0, 0)),
            pl.BlockSpec((1, _E * _HID, 1), _bmap),
            pl.BlockSpec((N, _E, _HID, _D), lambda p: (0, 0, 0, 0)),
            pl.BlockSpec((1, _E * _D, 1), _bmap),
        ],
        out_specs=[
            pl.BlockSpec((1, 1), lambda p: (0, 0),
                         memory_space=pltpu.SMEM),
            pl.BlockSpec((1, B), lambda p: (0, 0)),
        ],
        out_shape=[
            jax.ShapeDtypeStruct((1, 1), jnp.float32),
            jax.ShapeDtypeStruct((1, B), jnp.int32),
        ],
        scratch_shapes=[
            pltpu.VMEM((_TLM + _TOUT - 1, _N, _D, _B), jnp.bfloat16),
            pltpu.VMEM((_TOUT, _N, _D, _B), jnp.float32),
            pltpu.VMEM((_N, _E * _HID, _IN), jnp.bfloat16),
            pltpu.VMEM((_N, _E * _D, _E * _HID), jnp.bfloat16),
        ],
        compiler_params=pltpu.CompilerParams(
            vmem_limit_bytes=100 * 1024 * 1024,
        ),
    )(xf, W1, b1r, W2, b2r)

    return (total[0, 0], eidx[0])


# speedup vs baseline: 1.3132x; 1.3132x over previous
import jax
import jax.numpy as jnp
from jax.experimental import pallas as pl
from jax.experimental.pallas import tpu as pltpu

_N = 25
_E = 8
_D = 16
_TLM = 5
_NBR = 3
_HID = 32
_TOUT = 3
_T = 8
_B = 1024


def _moe_body(xf, w1, b1r, w2n, b2r, tot_ref, eidx_ref, xbf_ref, targ_ref, w2b_ref):
    p = pl.program_id(0)
    f32 = jnp.float32
    B = eidx_ref.shape[-1]

    @pl.when(p == 0)
    def _transpose():
        big = xf[...].T.reshape(_T, _N, _D, B)
        targ_ref[...] = big[2:2 + _TOUT]
        xbf_ref[...] = big[:_TLM + _TOUT - 1].astype(jnp.bfloat16)
        w2b_ref[...] = jnp.zeros(w2b_ref.shape, jnp.bfloat16)
        w2t = jnp.transpose(w2n[...], (0, 1, 3, 2)).astype(jnp.bfloat16)
        for e in range(_E):
            w2b_ref[:, _D * e:_D * (e + 1),
                    _HID * e:_HID * (e + 1)] = w2t[:, e]
        tot_ref[0, 0] = f32(0.0)

    @pl.when(p > 0)
    def _compute():
        j = p - 1
        slab = {(tt, nb): xbf_ref[tt, (j + nb + _N - 1) % _N]
                for tt in range(_TLM + _TOUT - 1) for nb in range(_NBR)}
        flat3 = jnp.concatenate(
            [jnp.concatenate([slab[(tc + tl, nb)] for tl in range(_TLM)
                              for nb in range(_NBR)], axis=0)
             for tc in range(_TOUT)], axis=1)

        h = jnp.maximum(
            jnp.dot(w1[0], flat3, preferred_element_type=f32) + b1r[0],
            0.0)
        pred = jnp.dot(w2b_ref[j], h.astype(jnp.bfloat16),
                       preferred_element_type=f32) + b2r[0]

        targ = jnp.concatenate([targ_ref[tc, j] for tc in range(_TOUT)],
                               axis=1)
        targ_t = jnp.concatenate([targ] * _E, axis=0)
        sq = (pred - targ_t) * (pred - targ_t)

        err = jnp.sum(sq.reshape(_E, _D, sq.shape[-1]), axis=1) * f32(1. / _D)

        minv = jnp.min(err, axis=0, keepdims=True)
        tot_ref[0, 0] = tot_ref[0, 0] + jnp.sum(minv)

        @pl.when(j == _N - 1)
        def _final():
            err_l = err[:, (_TOUT - 1) * B:]
            min_l = minv[:, (_TOUT - 1) * B:]
            ei = jax.lax.broadcasted_iota(jnp.int32, err_l.shape, 0)
            amin = jnp.min(jnp.where(err_l == min_l, ei, _E), axis=0)
            eidx_ref[0, :] = amin
            tot_ref[0, 0] = tot_ref[0, 0] * f32(1.0 / (B * (_N - 1) * 4))


def kernel(x, W1, b1, W2, b2):
    B, T, N, D = x.shape
    in_dim = _TLM * _NBR * _D
    bf = jnp.bfloat16

    xf = x.reshape(B, T * N * D)

    W1t = W1.transpose(0, 1, 3, 2).reshape(N, _E * _HID, in_dim).astype(bf)
    b1r = b1.reshape(N, _E * _HID, 1)
    b2r = b2.reshape(N, _E * _D, 1)

    def _wmap(p):
        return (jnp.maximum(p - 1, 0), 0, 0)

    total, eidx = pl.pallas_call(
        _moe_body,
        grid=(1 + N,),
        in_specs=[
            pl.BlockSpec((B, T * N * D), lambda p: (0, 0)),
            pl.BlockSpec((1, _E * _HID, in_dim), _wmap),
            pl.BlockSpec((1, _E * _HID, 1), _wmap),
            pl.BlockSpec((N, _E, _HID, _D), lambda p: (0, 0, 0, 0)),
            pl.BlockSpec((1, _E * _D, 1), _wmap),
        ],
        out_specs=[
            pl.BlockSpec((1, 1), lambda p: (0, 0),
                         memory_space=pltpu.SMEM),
            pl.BlockSpec((1, B), lambda p: (0, 0)),
        ],
        out_shape=[
            jax.ShapeDtypeStruct((1, 1), jnp.float32),
            jax.ShapeDtypeStruct((1, B), jnp.int32),
        ],
        scratch_shapes=[
            pltpu.VMEM((_TLM + _TOUT - 1, _N, _D, _B), bf),
            pltpu.VMEM((_TOUT, _N, _D, _B), jnp.float32),
            pltpu.VMEM((_N, _E * _D, _E * _HID), bf),
        ],
        compiler_params=pltpu.CompilerParams(
            vmem_limit_bytes=100 * 1024 * 1024,
        ),
    )(xf, W1t, b1r, W2, b2r)

    return (total[0, 0], eidx[0])
